# baseline (speedup 1.0000x reference)
import jax
import jax.numpy as jnp
from jax.experimental import pallas as pl
from jax.experimental.pallas import tpu as pltpu

ISIZE = 224
PSIZE = 30
WIN = 40
C = 8
K = 4
LAT = 2


def _win_tile(pos_ref, pf_ref, s):
    py = pos_ref[s, 0]
    px = pos_ref[s, 1]
    ry = pl.multiple_of(jnp.minimum((py // 8) * 8, ISIZE - WIN), 8)
    tile = pltpu.roll(pf_ref[0], px, axis=2)
    return ry, pltpu.roll(tile, py - ry, axis=1)


def _row0(pos_ref, s):
    py = pos_ref[s, 0]
    return pl.multiple_of(jnp.minimum((py // 8) * 8, ISIZE - WIN), 8)


def _make_kernel(B):
    N = B // C

    def body(pos_ref, x_hbm, pf_ref, out_hbm, wbuf, csem, rsem, wsem):
        t = pl.program_id(0)

        def cp_copy(c):
            k = jax.lax.rem(c, K)
            return pltpu.make_async_copy(
                x_hbm.at[pl.ds(c * C, C)],
                out_hbm.at[pl.ds(c * C, C)],
                csem.at[k],
            )

        def rd_copy(c, i):
            k = jax.lax.rem(c, K)
            b = c * C + i
            ry = _row0(pos_ref, b)
            return pltpu.make_async_copy(
                x_hbm.at[b, :, pl.ds(ry, WIN), :],
                wbuf.at[k, i],
                rsem.at[k, i],
            )

        def wr_copy(c, i):
            k = jax.lax.rem(c, K)
            b = c * C + i
            ry = _row0(pos_ref, b)
            return pltpu.make_async_copy(
                wbuf.at[k, i],
                out_hbm.at[b, :, pl.ds(ry, WIN), :],
                wsem.at[k, i],
            )

        @pl.when(t < N)
        def _():
            @pl.when(t >= K)
            def _():
                for i in range(C):
                    wr_copy(t - K, i).wait()

            cp_copy(t).start()
            for i in range(C):
                rd_copy(t, i).start()

        s = t - LAT

        @pl.when((s >= 0) & (s < N))
        def _():
            cp_copy(s).wait()
            k = jax.lax.rem(s, K)
            for i in range(C):
                rd_copy(s, i).wait()
                _, tile = _win_tile(pos_ref, pf_ref, s * C + i)
                wbuf[k, i] = wbuf[k, i] + tile
                wr_copy(s, i).start()

        @pl.when(t == N + LAT - 1)
        def _():
            for j in range(K):
                for i in range(C):
                    wr_copy(N - K + j, i).wait()

    return body, N


def kernel(x, patch, pos):
    B = x.shape[0]
    patch_pad = jnp.zeros((1, 3, WIN, ISIZE), dtype=patch.dtype)
    patch_pad = jax.lax.dynamic_update_slice(patch_pad, patch, (0, 0, 0, 0))
    body, N = _make_kernel(B)
    grid_spec = pltpu.PrefetchScalarGridSpec(
        num_scalar_prefetch=1,
        grid=(N + LAT,),
        in_specs=[
            pl.BlockSpec(memory_space=pl.ANY),
            pl.BlockSpec((1, 3, WIN, ISIZE), lambda t, pos_ref: (0, 0, 0, 0)),
        ],
        out_specs=pl.BlockSpec(memory_space=pl.ANY),
        scratch_shapes=[
            pltpu.VMEM((K, C, 3, WIN, ISIZE), jnp.float32),
            pltpu.SemaphoreType.DMA((K,)),
            pltpu.SemaphoreType.DMA((K, C)),
            pltpu.SemaphoreType.DMA((K, C)),
        ],
    )
    return pl.pallas_call(
        body,
        grid_spec=grid_spec,
        out_shape=jax.ShapeDtypeStruct(x.shape, x.dtype),
    )(pos, x, patch_pad)


# speedup vs baseline: 13.3769x; 13.3769x over previous
import jax
import jax.numpy as jnp
from jax.experimental import pallas as pl
from jax.experimental.pallas import tpu as pltpu

ISIZE = 224
PSIZE = 30
WIN = 40
C = 16
K = 4
LAT = 2
SPLIT = 2


def _win_tile(pos_ref, pf_ref, s):
    py = pos_ref[s, 0]
    px = pos_ref[s, 1]
    ry = pl.multiple_of(jnp.minimum((py // 8) * 8, ISIZE - WIN), 8)
    tile = pltpu.roll(pf_ref[0], px, axis=2)
    return ry, pltpu.roll(tile, py - ry, axis=1)


def _make_kernel(B):
    N = B // C
    H = C // SPLIT

    def body(pos_ref, x_hbm, pf_ref, out_hbm, rbuf, rsem, wsem):
        t = pl.program_id(0)

        def rd_copies(c):
            k = jax.lax.rem(c, K)
            return [
                pltpu.make_async_copy(
                    x_hbm.at[pl.ds(c * C + j * H, H)],
                    rbuf.at[pl.ds(k * C + j * H, H)],
                    rsem.at[k, j],
                )
                for j in range(SPLIT)
            ]

        def wr_copies(c):
            k = jax.lax.rem(c, K)
            return [
                pltpu.make_async_copy(
                    rbuf.at[pl.ds(k * C + j * H, H)],
                    out_hbm.at[pl.ds(c * C + j * H, H)],
                    wsem.at[k, j],
                )
                for j in range(SPLIT)
            ]

        @pl.when(t < N)
        def _():
            @pl.when(t >= K)
            def _():
                for cp in wr_copies(t - K):
                    cp.wait()

            for cp in rd_copies(t):
                cp.start()

        s = t - LAT

        @pl.when((s >= 0) & (s < N))
        def _():
            for cp in rd_copies(s):
                cp.wait()
            k = jax.lax.rem(s, K)
            for i in range(C):
                b = s * C + i
                ry, tile = _win_tile(pos_ref, pf_ref, b)
                row = k * C + i
                rbuf[row, :, pl.ds(ry, WIN), :] = (
                    rbuf[row, :, pl.ds(ry, WIN), :] + tile
                )
            for cp in wr_copies(s):
                cp.start()

        @pl.when(t == N + LAT - 1)
        def _():
            for j in range(K):
                for cp in wr_copies(N - K + j):
                    cp.wait()

    return body, N


def kernel(x, patch, pos):
    B = x.shape[0]
    patch_pad = jnp.zeros((1, 3, WIN, ISIZE), dtype=patch.dtype)
    patch_pad = jax.lax.dynamic_update_slice(patch_pad, patch, (0, 0, 0, 0))
    body, N = _make_kernel(B)
    grid_spec = pltpu.PrefetchScalarGridSpec(
        num_scalar_prefetch=1,
        grid=(N + LAT,),
        in_specs=[
            pl.BlockSpec(memory_space=pl.ANY),
            pl.BlockSpec((1, 3, WIN, ISIZE), lambda t, pos_ref: (0, 0, 0, 0)),
        ],
        out_specs=pl.BlockSpec(memory_space=pl.ANY),
        scratch_shapes=[
            pltpu.VMEM((K * C, 3, ISIZE, ISIZE), jnp.float32),
            pltpu.SemaphoreType.DMA((K, SPLIT)),
            pltpu.SemaphoreType.DMA((K, SPLIT)),
        ],
    )
    return pl.pallas_call(
        body,
        grid_spec=grid_spec,
        out_shape=jax.ShapeDtypeStruct(x.shape, x.dtype),
    )(pos, x, patch_pad)


# speedup vs baseline: 15.4493x; 1.1549x over previous
import jax
import jax.numpy as jnp
from jax.experimental import pallas as pl
from jax.experimental.pallas import tpu as pltpu

ISIZE = 224
PSIZE = 30
WIN = 40
C = 8
K = 4
LAT = 2


def _win_tile(pos_ref, pf_ref, s):
    py = pos_ref[s, 0]
    px = pos_ref[s, 1]
    ry = pl.multiple_of(jnp.minimum((py // 8) * 8, ISIZE - WIN), 8)
    tile = pltpu.roll(pf_ref[0], px, axis=2)
    return ry, pltpu.roll(tile, py - ry, axis=1)


def _row0(pos_ref, s):
    py = pos_ref[s, 0]
    return pl.multiple_of(jnp.minimum((py // 8) * 8, ISIZE - WIN), 8)


def _make_kernel(B):
    N = B // C

    def body(pos_ref, x_hbm, pf_ref, out_hbm, wbuf, rsem, wsem):
        t = pl.program_id(0)

        def rd_copy(c, i):
            k = jax.lax.rem(c, K)
            b = c * C + i
            ry = _row0(pos_ref, b)
            return pltpu.make_async_copy(
                out_hbm.at[b, :, pl.ds(ry, WIN), :],
                wbuf.at[k, i],
                rsem.at[k, i],
            )

        def wr_copy(c, i):
            k = jax.lax.rem(c, K)
            b = c * C + i
            ry = _row0(pos_ref, b)
            return pltpu.make_async_copy(
                wbuf.at[k, i],
                out_hbm.at[b, :, pl.ds(ry, WIN), :],
                wsem.at[k, i],
            )

        @pl.when(t < N)
        def _():
            @pl.when(t >= K)
            def _():
                for i in range(C):
                    wr_copy(t - K, i).wait()

            for i in range(C):
                rd_copy(t, i).start()

        s = t - LAT

        @pl.when((s >= 0) & (s < N))
        def _():
            k = jax.lax.rem(s, K)
            for i in range(C):
                rd_copy(s, i).wait()
                _, tile = _win_tile(pos_ref, pf_ref, s * C + i)
                wbuf[k, i] = wbuf[k, i] + tile
                wr_copy(s, i).start()

        @pl.when(t == N + LAT - 1)
        def _():
            for j in range(K):
                for i in range(C):
                    wr_copy(N - K + j, i).wait()

    return body, N


def kernel(x, patch, pos):
    B = x.shape[0]
    patch_pad = jnp.zeros((1, 3, WIN, ISIZE), dtype=patch.dtype)
    patch_pad = jax.lax.dynamic_update_slice(patch_pad, patch, (0, 0, 0, 0))
    body, N = _make_kernel(B)
    grid_spec = pltpu.PrefetchScalarGridSpec(
        num_scalar_prefetch=1,
        grid=(N + LAT,),
        in_specs=[
            pl.BlockSpec(memory_space=pl.ANY),
            pl.BlockSpec((1, 3, WIN, ISIZE), lambda t, pos_ref: (0, 0, 0, 0)),
        ],
        out_specs=pl.BlockSpec(memory_space=pl.ANY),
        scratch_shapes=[
            pltpu.VMEM((K, C, 3, WIN, ISIZE), jnp.float32),
            pltpu.SemaphoreType.DMA((K, C)),
            pltpu.SemaphoreType.DMA((K, C)),
        ],
    )
    return pl.pallas_call(
        body,
        grid_spec=grid_spec,
        out_shape=jax.ShapeDtypeStruct(x.shape, x.dtype),
        input_output_aliases={1: 0},
    )(pos, x, patch_pad)


# speedup vs baseline: 15.4631x; 1.0009x over previous
import jax
import jax.numpy as jnp
from jax.experimental import pallas as pl
from jax.experimental.pallas import tpu as pltpu

ISIZE = 224
PSIZE = 30
WIN = 40
C = 16
K = 4
LAT = 2


def _win_tile(pos_ref, pf_ref, s):
    py = pos_ref[s, 0]
    px = pos_ref[s, 1]
    ry = pl.multiple_of(jnp.minimum((py // 8) * 8, ISIZE - WIN), 8)
    tile = pltpu.roll(pf_ref[0], px, axis=2)
    return ry, pltpu.roll(tile, py - ry, axis=1)


def _row0(pos_ref, s):
    py = pos_ref[s, 0]
    return pl.multiple_of(jnp.minimum((py // 8) * 8, ISIZE - WIN), 8)


def _make_kernel(B):
    N = B // C

    def body(pos_ref, x_hbm, pf_ref, out_hbm, wbuf, rsem, wsem):
        t = pl.program_id(0)

        def rd_copy(c, i):
            k = jax.lax.rem(c, K)
            b = c * C + i
            ry = _row0(pos_ref, b)
            return pltpu.make_async_copy(
                out_hbm.at[b, :, pl.ds(ry, WIN), :],
                wbuf.at[k, i],
                rsem.at[k, i],
            )

        def wr_copy(c, i):
            k = jax.lax.rem(c, K)
            b = c * C + i
            ry = _row0(pos_ref, b)
            return pltpu.make_async_copy(
                wbuf.at[k, i],
                out_hbm.at[b, :, pl.ds(ry, WIN), :],
                wsem.at[k, i],
            )

        @pl.when(t < N)
        def _():
            @pl.when(t >= K)
            def _():
                for i in range(C):
                    wr_copy(t - K, i).wait()

            for i in range(C):
                rd_copy(t, i).start()

        s = t - LAT

        @pl.when((s >= 0) & (s < N))
        def _():
            k = jax.lax.rem(s, K)
            for i in range(C):
                rd_copy(s, i).wait()
                _, tile = _win_tile(pos_ref, pf_ref, s * C + i)
                wbuf[k, i] = wbuf[k, i] + tile
                wr_copy(s, i).start()

        @pl.when(t == N + LAT - 1)
        def _():
            for j in range(K):
                for i in range(C):
                    wr_copy(N - K + j, i).wait()

    return body, N


def kernel(x, patch, pos):
    B = x.shape[0]
    patch_pad = jnp.zeros((1, 3, WIN, ISIZE), dtype=patch.dtype)
    patch_pad = jax.lax.dynamic_update_slice(patch_pad, patch, (0, 0, 0, 0))
    body, N = _make_kernel(B)
    grid_spec = pltpu.PrefetchScalarGridSpec(
        num_scalar_prefetch=1,
        grid=(N + LAT,),
        in_specs=[
            pl.BlockSpec(memory_space=pl.ANY),
            pl.BlockSpec((1, 3, WIN, ISIZE), lambda t, pos_ref: (0, 0, 0, 0)),
        ],
        out_specs=pl.BlockSpec(memory_space=pl.ANY),
        scratch_shapes=[
            pltpu.VMEM((K, C, 3, WIN, ISIZE), jnp.float32),
            pltpu.SemaphoreType.DMA((K, C)),
            pltpu.SemaphoreType.DMA((K, C)),
        ],
    )
    return pl.pallas_call(
        body,
        grid_spec=grid_spec,
        out_shape=jax.ShapeDtypeStruct(x.shape, x.dtype),
        input_output_aliases={1: 0},
    )(pos, x, patch_pad)
